# Initial kernel scaffold; baseline (speedup 1.0000x reference)
import jax
import jax.numpy as jnp
from jax.experimental import pallas as pl


def kernel(points, params):
    raise NotImplementedError("write your pallas kernel here")


# speedup vs baseline: 3.4633x; 3.4633x over previous
import functools

import jax
import jax.numpy as jnp
from jax import lax
from jax.experimental import pallas as pl
from jax.experimental.pallas import tpu as pltpu
from jax.experimental.pallas import tpu_sc as plsc


def _fps_body(x_ref, y_ref, z_ref, o_ref, *, m, n):
    x = x_ref[0]
    y = y_ref[0]
    z = z_ref[0]
    iota = jax.lax.broadcasted_iota(jnp.int32, (1, n), 1)
    miota = jax.lax.broadcasted_iota(jnp.int32, (1, m), 1)

    def step(i, carry):
        dist, far, idxs = carry
        idxs = jnp.where(miota == i, far, idxs)
        oh = iota == far
        cx = jnp.sum(jnp.where(oh, x, 0.0))
        cy = jnp.sum(jnp.where(oh, y, 0.0))
        cz = jnp.sum(jnp.where(oh, z, 0.0))
        d = (x - cx) ** 2 + (y - cy) ** 2 + (z - cz) ** 2
        dist = jnp.minimum(dist, d)
        mx = jnp.max(dist)
        far = jnp.min(jnp.where(dist == mx, iota, n))
        return dist, far, idxs

    dist0 = jnp.full((1, n), 1e10, jnp.float32)
    idxs0 = jnp.zeros((1, m), jnp.int32)
    _, _, idxs = jax.lax.fori_loop(
        0, m, step, (dist0, jnp.array(0, jnp.int32), idxs0))
    o_ref[0] = idxs


def _fps(xyz, m):
    bn, _, n = xyz.shape
    xs = xyz[:, 0:1, :]
    ys = xyz[:, 1:2, :]
    zs = xyz[:, 2:3, :]
    out = pl.pallas_call(
        functools.partial(_fps_body, m=m, n=n),
        grid=(bn,),
        in_specs=[pl.BlockSpec((1, 1, n), lambda b: (b, 0, 0))] * 3,
        out_specs=pl.BlockSpec((1, 1, m), lambda b: (b, 0, 0)),
        out_shape=jax.ShapeDtypeStruct((bn, 1, m), jnp.int32),
        compiler_params=pltpu.CompilerParams(
            dimension_semantics=("parallel",)),
    )(xs, ys, zs)
    return out[:, 0, :]


def _mlp_body(*refs, nl, pool):
    x_ref = refs[0]
    o_ref = refs[1 + 3 * nl]
    x = x_ref[0]
    for li in range(nl):
        wt = refs[1 + 3 * li][...]
        s = refs[2 + 3 * li][...]
        b = refs[3 + 3 * li][...]
        x = jnp.dot(x, wt, preferred_element_type=jnp.float32)
        x = jnp.maximum(x * s + b, 0.0)
    if pool is None:
        o_ref[0] = x
    elif pool == "all":
        o_ref[0] = jnp.max(x, axis=0, keepdims=True)
    else:
        k, m = pool
        acc = x[0:m]
        for j in range(1, k):
            acc = jnp.maximum(acc, x[j * m:(j + 1) * m])
        o_ref[0] = acc


def _run_mlp(x, layers, pool=None):
    bn, p, _ = x.shape
    cout = layers[-1][0].shape[1]
    if pool is None:
        rows = p
    elif pool == "all":
        rows = 1
    else:
        rows = pool[1]
    args = [x]
    specs = [pl.BlockSpec((1, p, x.shape[2]), lambda b: (b, 0, 0))]
    for (wt, s, bb) in layers:
        for a in (wt, s, bb):
            args.append(a)
            specs.append(pl.BlockSpec(a.shape, lambda b: (0, 0)))
    out = pl.pallas_call(
        functools.partial(_mlp_body, nl=len(layers), pool=pool),
        grid=(bn,),
        in_specs=specs,
        out_specs=pl.BlockSpec((1, rows, cout), lambda b: (b, 0, 0)),
        out_shape=jax.ShapeDtypeStruct((bn, rows, cout), jnp.float32),
        compiler_params=pltpu.CompilerParams(
            dimension_semantics=("parallel",)),
    )(*args)
    return out


def _prep(layers):
    out = []
    for (w, g, b) in layers:
        s = (g / jnp.sqrt(1.0 + 1e-5)).reshape(1, -1)
        out.append((w.T, s, b.reshape(1, -1)))
    return out


def _bq_sc_body(x_hbm, y_hbm, z_hbm, cx_hbm, cy_hbm, cz_hbm, out_hbm,
                xv, yv, zv, cxv, cyv, czv, buf, *, n, mloc, k, r2):
    wid = lax.axis_index("s") * 2 + lax.axis_index("c")
    b = wid // 2
    h = wid % 2
    pltpu.sync_copy(x_hbm.at[b], xv)
    pltpu.sync_copy(y_hbm.at[b], yv)
    pltpu.sync_copy(z_hbm.at[b], zv)
    pltpu.sync_copy(cx_hbm.at[b, pl.ds(h * mloc, mloc)], cxv)
    pltpu.sync_copy(cy_hbm.at[b, pl.ds(h * mloc, mloc)], cyv)
    pltpu.sync_copy(cz_hbm.at[b, pl.ds(h * mloc, mloc)], czv)
    lane = lax.iota(jnp.int32, 16)
    nchunk = n // 16

    def per_chunk(t, carry):
        cvx = cxv[pl.ds(t * 16, 16)]
        cvy = cyv[pl.ds(t * 16, 16)]
        cvz = czv[pl.ds(t * 16, 16)]
        for ci in range(16):
            c = t * 16 + ci
            ccx = cvx[ci]
            ccy = cvy[ci]
            ccz = cvz[ci]
            row = jnp.full((16,), c, jnp.int32)

            def cond(st):
                j, cnt = st
                return (j < nchunk) & (cnt < k)

            def body(st):
                j, cnt = st
                sl = pl.ds(j * 16, 16)
                dx = xv[sl] - ccx
                dy = yv[sl] - ccy
                dz = zv[sl] - ccz
                d2 = dx * dx + dy * dy + dz * dz
                msk = d2 < r2
                pc = plsc.cumsum(msk.astype(jnp.int32))
                pos = cnt + pc - 1
                ok = msk & (pos < k)
                plsc.store_scatter(buf, [row, pos], lane + j * 16, mask=ok)
                return j + 1, cnt + jnp.max(pc)

            _, cnt = lax.while_loop(
                cond, body, (jnp.int32(0), jnp.int32(0)))
            first = jnp.where(cnt > 0, buf[c, pl.ds(0, 16)][0], 0)
            for t2 in range(k // 16):
                sl = pl.ds(t2 * 16, 16)
                v = buf[c, sl]
                gi = lane + t2 * 16
                buf[c, sl] = jnp.where(gi < cnt, v, first)
        return carry

    lax.fori_loop(0, mloc // 16, per_chunk, 0)
    pltpu.sync_copy(buf, out_hbm.at[b, pl.ds(h * mloc, mloc)])


def _bq_sc(xyz, new_xyz, radius, k):
    bn, _, n = xyz.shape
    mtot = new_xyz.shape[2]
    mloc = mtot // 2
    import functools as _ft
    mesh = plsc.VectorSubcoreMesh(core_axis_name="c", subcore_axis_name="s")
    kfn = _ft.partial(
        pl.kernel,
        mesh=mesh,
        out_type=jax.ShapeDtypeStruct((bn, mtot, k), jnp.int32),
        scratch_types=[
            pltpu.VMEM((n,), jnp.float32),
            pltpu.VMEM((n,), jnp.float32),
            pltpu.VMEM((n,), jnp.float32),
            pltpu.VMEM((mloc,), jnp.float32),
            pltpu.VMEM((mloc,), jnp.float32),
            pltpu.VMEM((mloc,), jnp.float32),
            pltpu.VMEM((mloc, k), jnp.int32),
        ],
    )(_ft.partial(_bq_sc_body, n=n, mloc=mloc, k=k,
                  r2=float(radius * radius)))
    return kfn(xyz[:, 0, :], xyz[:, 1, :], xyz[:, 2, :],
               new_xyz[:, 0, :], new_xyz[:, 1, :], new_xyz[:, 2, :])


def _interp_body(st_ref, dt_ref, sf_ref, o_ref, *, ns, nd):
    st = st_ref[0]
    dt = dt_ref[0]
    sf = sf_ref[0]
    d2 = jnp.zeros((ns, nd), jnp.float32)
    for a in range(3):
        diff = dt[a:a + 1, :] - st[a:a + 1, :].T
        d2 = d2 + diff * diff
    riota = jax.lax.broadcasted_iota(jnp.int32, (ns, nd), 0)
    wmat = jnp.zeros((ns, nd), jnp.float32)
    invs = []
    sel = []
    for _ in range(3):
        m1 = jnp.min(d2, axis=0, keepdims=True)
        i1 = jnp.min(jnp.where(d2 == m1, riota, ns), axis=0,
                     keepdims=True)
        oh = riota == i1
        invs.append(1.0 / jnp.maximum(m1, 1e-10))
        sel.append(oh)
        d2 = jnp.where(oh, jnp.inf, d2)
    tot = invs[0] + invs[1] + invs[2]
    for oh, inv in zip(sel, invs):
        wmat = wmat + jnp.where(oh, inv / tot, 0.0)
    o_ref[0] = jnp.dot(sf, wmat, preferred_element_type=jnp.float32)


def _interp_tc(dense_xyz, sparse_xyz, sparse_feature):
    bn, _, nd = dense_xyz.shape
    ns = sparse_xyz.shape[2]
    c = sparse_feature.shape[1]
    return pl.pallas_call(
        functools.partial(_interp_body, ns=ns, nd=nd),
        grid=(bn,),
        in_specs=[
            pl.BlockSpec((1, 3, ns), lambda b: (b, 0, 0)),
            pl.BlockSpec((1, 3, nd), lambda b: (b, 0, 0)),
            pl.BlockSpec((1, c, ns), lambda b: (b, 0, 0)),
        ],
        out_specs=pl.BlockSpec((1, c, nd), lambda b: (b, 0, 0)),
        out_shape=jax.ShapeDtypeStruct((bn, c, nd), jnp.float32),
        compiler_params=pltpu.CompilerParams(
            dimension_semantics=("arbitrary",)),
    )(sparse_xyz, dense_xyz, sparse_feature)


def _g(feat, idx):
    bn, c = feat.shape[0], feat.shape[1]
    flat = idx.reshape(bn, 1, -1)
    g = jnp.take_along_axis(
        feat, jnp.broadcast_to(flat, (bn, c, flat.shape[2])), axis=2)
    return g.reshape((bn, c) + idx.shape[1:])


def _bq(xyz_t, new_xyz_t, radius, k):
    bn, n, _ = xyz_t.shape
    d2 = jnp.sum((new_xyz_t[:, :, None, :] - xyz_t[:, None, :, :]) ** 2,
                 axis=-1)
    mask = d2 < radius * radius
    ar = jnp.arange(n, dtype=jnp.int32)
    cand = jnp.where(mask, ar[None, None, :], n)
    neg, _ = jax.lax.top_k(-cand, k)
    cand = -neg
    first = cand[:, :, :1]
    first = jnp.where(first >= n, 0, first)
    return jnp.where(cand >= n, first, cand)


def _interp(dense_xyz, sparse_xyz, sparse_feature, k):
    dt = dense_xyz.transpose(0, 2, 1)
    st = sparse_xyz.transpose(0, 2, 1)
    d2 = jnp.sum((dt[:, :, None, :] - st[:, None, :, :]) ** 2, axis=-1)
    negv, idx = jax.lax.top_k(-d2, k)
    dist = jnp.maximum(-negv, 1e-10)
    inv = 1.0 / dist
    w = inv / jnp.sum(inv, axis=-1, keepdims=True)
    g = _g(sparse_feature, idx.astype(jnp.int32))
    return jnp.sum(g * w[:, None, :, :], axis=-1)


def _group_flat(g):
    return g.transpose(0, 3, 2, 1).reshape(g.shape[0], -1, g.shape[1])


def kernel(points, params):
    sa = [_prep(l) for l in params["sa"]]
    fp = [_prep(l) for l in params["fp"]]
    seg = _prep(params["seg"])

    xyz0 = points[:, 0:3, :]
    feat0 = points[:, 3:, :]
    bn = points.shape[0]

    cid1 = _fps(xyz0, 128)
    xyz1 = _g(xyz0, cid1)
    idx1 = _bq(xyz0.transpose(0, 2, 1), xyz1.transpose(0, 2, 1), 0.2, 64)
    g1 = jnp.concatenate(
        [_g(xyz0, idx1) - xyz1[:, :, :, None], _g(feat0, idx1)], axis=1)
    f1 = _run_mlp(_group_flat(g1), sa[0], pool=(64, 128))
    feat1 = f1.transpose(0, 2, 1)

    cid2 = _fps(xyz1, 32)
    xyz2 = _g(xyz1, cid2)
    idx2 = _bq(xyz1.transpose(0, 2, 1), xyz2.transpose(0, 2, 1), 0.4, 64)
    g2 = jnp.concatenate(
        [_g(xyz1, idx2) - xyz2[:, :, :, None], _g(feat1, idx2)], axis=1)
    f2 = _run_mlp(_group_flat(g2), sa[1], pool=(64, 32))
    feat2 = f2.transpose(0, 2, 1)

    g3 = jnp.concatenate([xyz2[:, :, None, :], feat2[:, :, None, :]], axis=1)
    x3 = g3[:, :, 0, :].transpose(0, 2, 1)
    f3 = _run_mlp(x3, sa[2], pool="all")
    feat3 = f3.transpose(0, 2, 1)

    exp = jnp.broadcast_to(feat3, (bn, 256, 32))
    nf = jnp.concatenate([exp, feat2], axis=1)
    o1 = _run_mlp(nf.transpose(0, 2, 1), fp[0])
    sfeat = o1.transpose(0, 2, 1)

    it = _interp_tc(xyz1, xyz2, sfeat)
    nf = jnp.concatenate([it, feat1], axis=1)
    o2 = _run_mlp(nf.transpose(0, 2, 1), fp[1])
    sfeat = o2.transpose(0, 2, 1)

    it = _interp_tc(xyz0, xyz1, sfeat)
    nf = jnp.concatenate([it, points], axis=1)
    o3 = _run_mlp(nf.transpose(0, 2, 1), fp[2] + seg)
    return o3.transpose(0, 2, 1)
